# Initial kernel scaffold; baseline (speedup 1.0000x reference)
import functools

import jax
import jax.numpy as jnp
from jax import lax
from jax.experimental import pallas as pl
from jax.experimental.pallas import tpu as pltpu
from jax.experimental.pallas import tpu_sc as plsc

N = 10000
E = 320000
H = 64

NC = 2
NS = 16
NW = NC * NS

GCH = 100
G_ROWS = 2 * E // GCH
G_PER_W = G_ROWS // NW
S_ROWS = E // GCH
S_PER_W = S_ROWS // NW
N_PER_S = N // NS

BE = 2560
NB = E // BE


def _act(x):
    return jnp.where(x > 0, x, 0.1 * x)


def _ln(x, g, b):
    mu = jnp.mean(x, -1, keepdims=True)
    d = x - mu
    var = jnp.mean(d * d, -1, keepdims=True)
    return d * lax.rsqrt(var + 1e-5) * g + b


def _sc_gather(table, idx2d):
    mesh = plsc.VectorSubcoreMesh(core_axis_name="c", subcore_axis_name="s")

    @functools.partial(
        pl.kernel,
        out_type=jax.ShapeDtypeStruct((2 * E, H), jnp.float32),
        mesh=mesh,
        scratch_types=[
            pltpu.VMEM((G_PER_W, GCH), jnp.int32),
            pltpu.VMEM((GCH, H), jnp.float32),
            pltpu.VMEM((GCH, H), jnp.float32),
            pltpu.SemaphoreType.DMA,
            pltpu.SemaphoreType.DMA,
        ],
    )
    def k(table_hbm, idx_hbm, out_hbm, idx_v, buf0, buf1, sem0, sem1):
        c = lax.axis_index("c")
        s = lax.axis_index("s")
        w = s * NC + c
        r0 = w * G_PER_W
        pltpu.sync_copy(idx_hbm.at[pl.ds(r0, G_PER_W)], idx_v)
        bufs = (buf0, buf1)
        sems = (sem0, sem1)
        pltpu.async_copy(table_hbm.at[idx_v.at[0]], buf0, sem0)

        def body(j, _):
            @pl.when(j + 1 < G_PER_W)
            def _():
                for b in range(2):
                    @pl.when((j + 1) % 2 == b)
                    def _():
                        pltpu.async_copy(
                            table_hbm.at[idx_v.at[j + 1]], bufs[b], sems[b])
            for b in range(2):
                @pl.when(j % 2 == b)
                def _():
                    pltpu.make_async_copy(
                        table_hbm.at[idx_v.at[j]], bufs[b], sems[b]).wait()
                    pltpu.sync_copy(
                        bufs[b], out_hbm.at[pl.ds((r0 + j) * GCH, GCH)])
            return 0

        lax.fori_loop(0, G_PER_W, body, 0)

    return k(table, idx2d)


def _sc_scatter_add(m2, idx2d, zeros):
    mesh = plsc.VectorSubcoreMesh(core_axis_name="c", subcore_axis_name="s")

    @functools.partial(
        pl.kernel,
        out_type=jax.ShapeDtypeStruct((NC, N, H), jnp.float32),
        mesh=mesh,
        scratch_types=[
            pltpu.VMEM((S_PER_W, GCH), jnp.int32),
            pltpu.VMEM((GCH, H), jnp.float32),
            pltpu.VMEM_SHARED((N, H), jnp.float32),
            pltpu.SemaphoreType.DMA,
        ],
    )
    def k(m2_hbm, idx_hbm, zeros_hbm, out_hbm, idx_v, rows_v, acc_sh, sem):
        c = lax.axis_index("c")
        s = lax.axis_index("s")
        w = s * NC + c
        pltpu.sync_copy(zeros_hbm.at[pl.ds(s * N_PER_S, N_PER_S)],
                        acc_sh.at[pl.ds(s * N_PER_S, N_PER_S)])
        plsc.subcore_barrier()
        r0 = w * S_PER_W
        pltpu.sync_copy(idx_hbm.at[pl.ds(r0, S_PER_W)], idx_v)

        def body(j, _):
            pltpu.sync_copy(m2_hbm.at[pl.ds((r0 + j) * GCH, GCH)], rows_v)
            pltpu.sync_copy(rows_v, acc_sh.at[idx_v.at[j]], add=True)
            return 0

        lax.fori_loop(0, S_PER_W, body, 0)
        plsc.subcore_barrier()
        pltpu.sync_copy(acc_sh.at[pl.ds(s * N_PER_S, N_PER_S)],
                        out_hbm.at[c, pl.ds(s * N_PER_S, N_PER_S)])

    return k(m2, idx2d, zeros)


def _enc_body(a_ref, w_ref, v_ref, o_ref):
    a = a_ref[...]
    u = jnp.dot(a, w_ref[...], preferred_element_type=jnp.float32) + v_ref[0]
    o_ref[...] = _act(_ln(u, v_ref[1], v_ref[2]))


def _tc_encode(a, w, vecs, blk):
    n = a.shape[0]
    return pl.pallas_call(
        _enc_body,
        grid=(n // blk,),
        in_specs=[
            pl.BlockSpec((blk, a.shape[1]), lambda i: (i, 0)),
            pl.BlockSpec(w.shape, lambda i: (0, 0)),
            pl.BlockSpec(vecs.shape, lambda i: (0, 0)),
        ],
        out_specs=pl.BlockSpec((blk, H), lambda i: (i, 0)),
        out_shape=jax.ShapeDtypeStruct((n, H), jnp.float32),
    )(a, w, vecs)


def _edge_body(with_msg, xs_ref, xd_ref, e_ref, wa_ref, wb_ref, v_ref,
               s_ref, *m2_ref):
    v = v_ref[...]
    hin = jnp.concatenate([xs_ref[...], xd_ref[...], e_ref[...]], axis=1)
    u = jnp.dot(hin, wa_ref[...], preferred_element_type=jnp.float32)
    h1 = _act(_ln(u[:, :H] + v[0], v[2], v[3]))
    if with_msg:
        m = _act(u[:, H:] + v[1])
        z = jnp.dot(jnp.concatenate([h1, m], axis=1), wb_ref[...],
                    preferred_element_type=jnp.float32)
        z1 = z[:, :H]
    else:
        z1 = jnp.dot(h1, wb_ref[...], preferred_element_type=jnp.float32)
    h2 = _act(_ln(z1 + v[6], v[4], v[5]))
    s = jnp.sum(h2 * v[8], axis=1) + v_ref[9, 0]
    s_ref[...] = s
    if with_msg:
        m2_ref[0][...] = (z[:, H:] + v[7]) * jax.nn.sigmoid(s)[:, None]


def _tc_edge(gcat, e, wa, wb, vecs, with_msg):
    kin = 3 * H
    wdim = 2 * H if with_msg else H
    out_shape = [jax.ShapeDtypeStruct((E,), jnp.float32)]
    out_specs = [pl.BlockSpec((BE,), lambda i: (i,))]
    if with_msg:
        out_shape.append(jax.ShapeDtypeStruct((E, H), jnp.float32))
        out_specs.append(pl.BlockSpec((BE, H), lambda i: (i, 0)))
    outs = pl.pallas_call(
        functools.partial(_edge_body, with_msg),
        grid=(NB,),
        in_specs=[
            pl.BlockSpec((BE, H), lambda i: (i, 0)),
            pl.BlockSpec((BE, H), lambda i: (i + NB, 0)),
            pl.BlockSpec((BE, H), lambda i: (i, 0)),
            pl.BlockSpec((kin, wdim), lambda i: (0, 0)),
            pl.BlockSpec((wdim, wdim), lambda i: (0, 0)),
            pl.BlockSpec((10, H), lambda i: (0, 0)),
        ],
        out_specs=out_specs,
        out_shape=out_shape,
    )(gcat, gcat, e, wa, wb, vecs)
    return outs if with_msg else (outs[0], None)


def _node_upd_body(x_ref, a0_ref, a1_ref, w1_ref, w2_ref, v_ref, o_ref):
    x = x_ref[...]
    agg = a0_ref[0] + a1_ref[0]
    u = jnp.dot(jnp.concatenate([x, agg], axis=1), w1_ref[...],
                preferred_element_type=jnp.float32)
    h = _act(_ln(u + v_ref[0], v_ref[1], v_ref[2]))
    o_ref[...] = x + jnp.dot(h, w2_ref[...],
                             preferred_element_type=jnp.float32) + v_ref[3]


def _tc_node_update(x, agg2, w1, w2, vecs):
    blk = 2500
    return pl.pallas_call(
        _node_upd_body,
        grid=(N // blk,),
        in_specs=[
            pl.BlockSpec((blk, H), lambda i: (i, 0)),
            pl.BlockSpec((1, blk, H), lambda i: (0, i, 0)),
            pl.BlockSpec((1, blk, H), lambda i: (1, i, 0)),
            pl.BlockSpec((2 * H, H), lambda i: (0, 0)),
            pl.BlockSpec((H, H), lambda i: (0, 0)),
            pl.BlockSpec(vecs.shape, lambda i: (0, 0)),
        ],
        out_specs=pl.BlockSpec((blk, H), lambda i: (i, 0)),
        out_shape=jax.ShapeDtypeStruct((N, H), jnp.float32),
    )(x, agg2, agg2, w1, w2, vecs)


def _node_cls_body(x_ref, w1_ref, v_ref, o_ref):
    t = _act(jnp.dot(x_ref[...], w1_ref[...],
                     preferred_element_type=jnp.float32) + v_ref[0, :H // 2])
    o_ref[...] = jnp.sum(t * v_ref[1, :H // 2], axis=1) + v_ref[2, 0]


def _tc_node_cls(x, w1, vecs):
    blk = 2500
    return pl.pallas_call(
        _node_cls_body,
        grid=(N // blk,),
        in_specs=[
            pl.BlockSpec((blk, H), lambda i: (i, 0)),
            pl.BlockSpec((H, H // 2), lambda i: (0, 0)),
            pl.BlockSpec(vecs.shape, lambda i: (0, 0)),
        ],
        out_specs=pl.BlockSpec((blk,), lambda i: (i,)),
        out_shape=jax.ShapeDtypeStruct((N,), jnp.float32),
    )(x, w1, vecs)


def _edge_weights(ep, npp):
    if npp is not None:
        wa = jnp.zeros((3 * H, 2 * H), jnp.float32)
        wa = wa.at[:, :H].set(ep['W1'])
        wa = wa.at[0:H, H:].set(npp['mW1'][0:H])
        wa = wa.at[2 * H:3 * H, H:].set(npp['mW1'][H:2 * H])
        wb = jnp.zeros((2 * H, 2 * H), jnp.float32)
        wb = wb.at[:H, :H].set(ep['W2']).at[H:, H:].set(npp['mW2'])
        mb1, mb2 = npp['mb1'], npp['mb2']
    else:
        wa = ep['W1']
        wb = ep['W2']
        mb1 = mb2 = jnp.zeros((H,), jnp.float32)
    vecs = jnp.stack([
        ep['b1'], mb1, ep['g1'], ep['bb1'], ep['g2'], ep['bb2'],
        ep['b2'], mb2, ep['W3'][:, 0],
        jnp.full((H,), ep['b3'][0], jnp.float32),
    ])
    return wa, wb, vecs


def kernel(node_features, edge_index, edge_attr, params):
    p = params
    src = edge_index[0].astype(jnp.int32)
    dst = edge_index[1].astype(jnp.int32)
    idx_g = jnp.concatenate([src, dst]).reshape(G_ROWS, GCH)
    idx_s = dst.reshape(S_ROWS, GCH)
    zeros = jnp.zeros((N, H), jnp.float32)

    ne = p['node_enc']
    x = _tc_encode(node_features, ne['W'],
                   jnp.stack([ne['b'], ne['g'], ne['bb']]), N)
    ee = p['edge_enc']
    e = _tc_encode(edge_attr, ee['W'],
                   jnp.stack([ee['b'], ee['g'], ee['bb']]), BE)

    inter = []
    for i in range(4):
        ep = p['edge_nets'][i]
        npp = p['node_nets'][i]
        wa, wb, vecs = _edge_weights(ep, npp)
        g = _sc_gather(x, idx_g)
        s, m2 = _tc_edge(g, e, wa, wb, vecs, True)
        inter.append(s)
        agg2 = _sc_scatter_add(m2, idx_s, zeros)
        nvecs = jnp.stack([npp['ub1'], npp['ug1'], npp['ugb1'], npp['ub2']])
        x = _tc_node_update(x, agg2, npp['uW1'], npp['uW2'], nvecs)

    wa, wb, vecs = _edge_weights(p['edge_cls'], None)
    g = _sc_gather(x, idx_g)
    fes, _ = _tc_edge(g, e, wa, wb, vecs, False)

    nc = p['node_cls']
    cvecs = jnp.zeros((3, H), jnp.float32)
    cvecs = cvecs.at[0, :H // 2].set(nc['b1'])
    cvecs = cvecs.at[1, :H // 2].set(nc['W2'][:, 0])
    cvecs = cvecs.at[2, 0].set(nc['b2'][0])
    ns = _tc_node_cls(x, nc['W1'], cvecs)
    return (fes, ns, inter)


# speedup vs baseline: 1.8475x; 1.8475x over previous
import functools

import jax
import jax.numpy as jnp
from jax import lax
from jax.experimental import pallas as pl
from jax.experimental.pallas import tpu as pltpu
from jax.experimental.pallas import tpu_sc as plsc

N = 10000
E = 320000
H = 64

NC = 2
NS = 16
NW = NC * NS

GCH = 80
G_PER_W = 2 * E // GCH // NW
S_PER_W = E // GCH // NW
N_PAD = 10240
N_PER_S = N_PAD // NS

BE = 2560
NB = E // BE


def _act(x):
    return jnp.where(x > 0, x, 0.1 * x)


def _ln(x, g, b):
    mu = jnp.mean(x, -1, keepdims=True)
    d = x - mu
    var = jnp.mean(d * d, -1, keepdims=True)
    return d * lax.rsqrt(var + 1e-5) * g + b


def _sc_gather(table, idx3d):
    mesh = plsc.VectorSubcoreMesh(core_axis_name="c", subcore_axis_name="s")

    @functools.partial(
        pl.kernel,
        out_type=jax.ShapeDtypeStruct((2 * E, H), jnp.float32),
        mesh=mesh,
        scratch_types=[
            pltpu.VMEM((G_PER_W, GCH), jnp.int32),
            pltpu.VMEM((GCH, H), jnp.float32),
            pltpu.VMEM((GCH, H), jnp.float32),
            pltpu.SemaphoreType.DMA,
            pltpu.SemaphoreType.DMA,
        ],
        compiler_params=pltpu.CompilerParams(use_tc_tiling_on_sc=False),
    )
    def k(table_hbm, idx_hbm, out_hbm, idx_v, buf0, buf1, sem0, sem1):
        c = lax.axis_index("c")
        s = lax.axis_index("s")
        w = s * NC + c
        r0 = w * G_PER_W
        pltpu.sync_copy(idx_hbm.at[w], idx_v)
        bufs = (buf0, buf1)
        sems = (sem0, sem1)
        pltpu.async_copy(table_hbm.at[idx_v.at[0]], buf0, sem0)

        def body(j, _):
            @pl.when(j + 1 < G_PER_W)
            def _():
                for b in range(2):
                    @pl.when((j + 1) % 2 == b)
                    def _():
                        pltpu.async_copy(
                            table_hbm.at[idx_v.at[j + 1]], bufs[b], sems[b])
            for b in range(2):
                @pl.when(j % 2 == b)
                def _():
                    pltpu.make_async_copy(
                        table_hbm.at[idx_v.at[j]], bufs[b], sems[b]).wait()
                    pltpu.sync_copy(
                        bufs[b], out_hbm.at[pl.ds((r0 + j) * GCH, GCH)])
            return 0

        lax.fori_loop(0, G_PER_W, body, 0)

    return k(table, idx3d)


def _sc_scatter_add(m2, idx3d, zeros):
    mesh = plsc.VectorSubcoreMesh(core_axis_name="c", subcore_axis_name="s")

    @functools.partial(
        pl.kernel,
        out_type=jax.ShapeDtypeStruct((NC, N_PAD, H), jnp.float32),
        mesh=mesh,
        scratch_types=[
            pltpu.VMEM((S_PER_W, GCH), jnp.int32),
            pltpu.VMEM((GCH, H), jnp.float32),
            pltpu.VMEM_SHARED((N_PAD, H), jnp.float32),
            pltpu.SemaphoreType.DMA,
        ],
        compiler_params=pltpu.CompilerParams(use_tc_tiling_on_sc=False),
    )
    def k(m2_hbm, idx_hbm, zeros_hbm, out_hbm, idx_v, rows_v, acc_sh, sem):
        c = lax.axis_index("c")
        s = lax.axis_index("s")
        w = s * NC + c
        pltpu.sync_copy(zeros_hbm.at[pl.ds(s * N_PER_S, N_PER_S)],
                        acc_sh.at[pl.ds(s * N_PER_S, N_PER_S)])
        plsc.subcore_barrier()
        r0 = w * S_PER_W
        pltpu.sync_copy(idx_hbm.at[w], idx_v)

        def body(j, _):
            pltpu.sync_copy(m2_hbm.at[pl.ds((r0 + j) * GCH, GCH)], rows_v)
            pltpu.sync_copy(rows_v, acc_sh.at[idx_v.at[j]], add=True)
            return 0

        lax.fori_loop(0, S_PER_W, body, 0)
        plsc.subcore_barrier()
        pltpu.sync_copy(acc_sh.at[pl.ds(s * N_PER_S, N_PER_S)],
                        out_hbm.at[c, pl.ds(s * N_PER_S, N_PER_S)])

    return k(m2, idx3d, zeros)


def _enc_body(a_ref, w_ref, v_ref, o_ref):
    a = a_ref[...]
    u = jnp.dot(a, w_ref[...], preferred_element_type=jnp.float32) + v_ref[0]
    o_ref[...] = _act(_ln(u, v_ref[1], v_ref[2]))


def _tc_encode(a, w, vecs, blk):
    n = a.shape[0]
    return pl.pallas_call(
        _enc_body,
        grid=(n // blk,),
        in_specs=[
            pl.BlockSpec((blk, a.shape[1]), lambda i: (i, 0)),
            pl.BlockSpec(w.shape, lambda i: (0, 0)),
            pl.BlockSpec(vecs.shape, lambda i: (0, 0)),
        ],
        out_specs=pl.BlockSpec((blk, H), lambda i: (i, 0)),
        out_shape=jax.ShapeDtypeStruct((n, H), jnp.float32),
    )(a, w, vecs)


def _edge_body(with_msg, xs_ref, xd_ref, e_ref, wa_ref, wb_ref, v_ref,
               s_ref, *m2_ref):
    v = v_ref[...]
    hin = jnp.concatenate([xs_ref[...], xd_ref[...], e_ref[...]], axis=1)
    u = jnp.dot(hin, wa_ref[...], preferred_element_type=jnp.float32)
    h1 = _act(_ln(u[:, :H] + v[0], v[2], v[3]))
    if with_msg:
        m = _act(u[:, H:] + v[1])
        z = jnp.dot(jnp.concatenate([h1, m], axis=1), wb_ref[...],
                    preferred_element_type=jnp.float32)
        z1 = z[:, :H]
    else:
        z1 = jnp.dot(h1, wb_ref[...], preferred_element_type=jnp.float32)
    h2 = _act(_ln(z1 + v[6], v[4], v[5]))
    s = jnp.sum(h2 * v[8], axis=1) + v_ref[9, 0]
    s_ref[...] = s[None, None, :]
    if with_msg:
        m2_ref[0][...] = (z[:, H:] + v[7]) * jax.nn.sigmoid(s)[:, None]


def _tc_edge(gcat, e, wa, wb, vecs, with_msg):
    kin = 3 * H
    wdim = 2 * H if with_msg else H
    out_shape = [jax.ShapeDtypeStruct((NB, 1, BE), jnp.float32)]
    out_specs = [pl.BlockSpec((1, 1, BE), lambda i: (i, 0, 0))]
    if with_msg:
        out_shape.append(jax.ShapeDtypeStruct((E, H), jnp.float32))
        out_specs.append(pl.BlockSpec((BE, H), lambda i: (i, 0)))
    outs = pl.pallas_call(
        functools.partial(_edge_body, with_msg),
        grid=(NB,),
        in_specs=[
            pl.BlockSpec((BE, H), lambda i: (i, 0)),
            pl.BlockSpec((BE, H), lambda i: (i + NB, 0)),
            pl.BlockSpec((BE, H), lambda i: (i, 0)),
            pl.BlockSpec((kin, wdim), lambda i: (0, 0)),
            pl.BlockSpec((wdim, wdim), lambda i: (0, 0)),
            pl.BlockSpec((10, H), lambda i: (0, 0)),
        ],
        out_specs=out_specs,
        out_shape=out_shape,
    )(gcat, gcat, e, wa, wb, vecs)
    s = outs[0].reshape(E)
    return (s, outs[1]) if with_msg else (s, None)


def _node_upd_body(x_ref, a0_ref, a1_ref, w1_ref, w2_ref, v_ref, o_ref):
    x = x_ref[...]
    agg = a0_ref[0] + a1_ref[0]
    u = jnp.dot(jnp.concatenate([x, agg], axis=1), w1_ref[...],
                preferred_element_type=jnp.float32)
    h = _act(_ln(u + v_ref[0], v_ref[1], v_ref[2]))
    o_ref[...] = x + jnp.dot(h, w2_ref[...],
                             preferred_element_type=jnp.float32) + v_ref[3]


def _tc_node_update(x, agg2, w1, w2, vecs):
    blk = 2000
    return pl.pallas_call(
        _node_upd_body,
        grid=(N // blk,),
        in_specs=[
            pl.BlockSpec((blk, H), lambda i: (i, 0)),
            pl.BlockSpec((1, blk, H), lambda i: (0, i, 0)),
            pl.BlockSpec((1, blk, H), lambda i: (1, i, 0)),
            pl.BlockSpec((2 * H, H), lambda i: (0, 0)),
            pl.BlockSpec((H, H), lambda i: (0, 0)),
            pl.BlockSpec(vecs.shape, lambda i: (0, 0)),
        ],
        out_specs=pl.BlockSpec((blk, H), lambda i: (i, 0)),
        out_shape=jax.ShapeDtypeStruct((N, H), jnp.float32),
    )(x, agg2, agg2, w1, w2, vecs)


def _node_cls_body(x_ref, w1_ref, v_ref, o_ref):
    t = _act(jnp.dot(x_ref[...], w1_ref[...],
                     preferred_element_type=jnp.float32) + v_ref[0, :H // 2])
    o_ref[...] = (jnp.sum(t * v_ref[1, :H // 2], axis=1)
                  + v_ref[2, 0])[None, None, :]


def _tc_node_cls(x, w1, vecs):
    blk = 2000
    return pl.pallas_call(
        _node_cls_body,
        grid=(N // blk,),
        in_specs=[
            pl.BlockSpec((blk, H), lambda i: (i, 0)),
            pl.BlockSpec((H, H // 2), lambda i: (0, 0)),
            pl.BlockSpec(vecs.shape, lambda i: (0, 0)),
        ],
        out_specs=pl.BlockSpec((1, 1, blk), lambda i: (i, 0, 0)),
        out_shape=jax.ShapeDtypeStruct((N // blk, 1, blk), jnp.float32),
    )(x, w1, vecs).reshape(N)


def _edge_weights(ep, npp):
    if npp is not None:
        wa = jnp.zeros((3 * H, 2 * H), jnp.float32)
        wa = wa.at[:, :H].set(ep['W1'])
        wa = wa.at[0:H, H:].set(npp['mW1'][0:H])
        wa = wa.at[2 * H:3 * H, H:].set(npp['mW1'][H:2 * H])
        wb = jnp.zeros((2 * H, 2 * H), jnp.float32)
        wb = wb.at[:H, :H].set(ep['W2']).at[H:, H:].set(npp['mW2'])
        mb1, mb2 = npp['mb1'], npp['mb2']
    else:
        wa = ep['W1']
        wb = ep['W2']
        mb1 = mb2 = jnp.zeros((H,), jnp.float32)
    vecs = jnp.stack([
        ep['b1'], mb1, ep['g1'], ep['bb1'], ep['g2'], ep['bb2'],
        ep['b2'], mb2, ep['W3'][:, 0],
        jnp.full((H,), ep['b3'][0], jnp.float32),
    ])
    return wa, wb, vecs


def kernel(node_features, edge_index, edge_attr, params):
    p = params
    src = edge_index[0].astype(jnp.int32)
    dst = edge_index[1].astype(jnp.int32)
    idx_g = jnp.concatenate([src, dst]).reshape(NW, G_PER_W, GCH)
    idx_s = dst.reshape(NW, S_PER_W, GCH)
    zeros = jnp.zeros((N_PAD, H), jnp.float32)

    ne = p['node_enc']
    x = _tc_encode(node_features, ne['W'],
                   jnp.stack([ne['b'], ne['g'], ne['bb']]), N)
    ee = p['edge_enc']
    e = _tc_encode(edge_attr, ee['W'],
                   jnp.stack([ee['b'], ee['g'], ee['bb']]), BE)

    inter = []
    for i in range(4):
        ep = p['edge_nets'][i]
        npp = p['node_nets'][i]
        wa, wb, vecs = _edge_weights(ep, npp)
        g = _sc_gather(x, idx_g)
        s, m2 = _tc_edge(g, e, wa, wb, vecs, True)
        inter.append(s)
        agg2 = _sc_scatter_add(m2, idx_s, zeros)
        nvecs = jnp.stack([npp['ub1'], npp['ug1'], npp['ugb1'], npp['ub2']])
        x = _tc_node_update(x, agg2, npp['uW1'], npp['uW2'], nvecs)

    wa, wb, vecs = _edge_weights(p['edge_cls'], None)
    g = _sc_gather(x, idx_g)
    fes, _ = _tc_edge(g, e, wa, wb, vecs, False)

    nc = p['node_cls']
    cvecs = jnp.zeros((3, H), jnp.float32)
    cvecs = cvecs.at[0, :H // 2].set(nc['b1'])
    cvecs = cvecs.at[1, :H // 2].set(nc['W2'][:, 0])
    cvecs = cvecs.at[2, 0].set(nc['b2'][0])
    ns = _tc_node_cls(x, nc['W1'], cvecs)
    return (fes, ns, inter)


# speedup vs baseline: 1.9702x; 1.0664x over previous
import functools

import jax
import jax.numpy as jnp
from jax import lax
from jax.experimental import pallas as pl
from jax.experimental.pallas import tpu as pltpu
from jax.experimental.pallas import tpu_sc as plsc

N = 10000
E = 320000
H = 64
E2 = E // 2

NC = 2
NS = 16
NW = NC * NS

GCH = 80
G_PER_W = 2 * E2 // GCH // NW
SCH = 40
S_PER_W = E2 // SCH // NW
N_PAD = 10240
N_PER_S = N_PAD // NS

BE = 2000
NB = E2 // BE


def _act(x):
    return jnp.where(x > 0, x, 0.1 * x)


def _ln(x, g, b):
    mu = jnp.mean(x, -1, keepdims=True)
    d = x - mu
    var = jnp.mean(d * d, -1, keepdims=True)
    return d * lax.rsqrt(var + 1e-5) * g + b


def _sc_gather(table, idx3d):
    mesh = plsc.VectorSubcoreMesh(core_axis_name="c", subcore_axis_name="s")

    @functools.partial(
        pl.kernel,
        out_type=jax.ShapeDtypeStruct((2 * E2, H), jnp.float32),
        mesh=mesh,
        scratch_types=[
            pltpu.VMEM((G_PER_W, GCH), jnp.int32),
            pltpu.VMEM((GCH, H), jnp.float32),
            pltpu.VMEM((GCH, H), jnp.float32),
            pltpu.SemaphoreType.DMA,
            pltpu.SemaphoreType.DMA,
        ],
        compiler_params=pltpu.CompilerParams(use_tc_tiling_on_sc=False),
    )
    def k(table_hbm, idx_hbm, out_hbm, idx_v, buf0, buf1, sem0, sem1):
        c = lax.axis_index("c")
        s = lax.axis_index("s")
        w = s * NC + c
        r0 = w * G_PER_W
        pltpu.sync_copy(idx_hbm.at[w], idx_v)
        bufs = (buf0, buf1)
        sems = (sem0, sem1)
        pltpu.async_copy(table_hbm.at[idx_v.at[0]], buf0, sem0)

        def body(j, _):
            @pl.when(j + 1 < G_PER_W)
            def _():
                for b in range(2):
                    @pl.when((j + 1) % 2 == b)
                    def _():
                        pltpu.async_copy(
                            table_hbm.at[idx_v.at[j + 1]], bufs[b], sems[b])
            for b in range(2):
                @pl.when(j % 2 == b)
                def _():
                    pltpu.make_async_copy(
                        table_hbm.at[idx_v.at[j]], bufs[b], sems[b]).wait()
                    pltpu.sync_copy(
                        bufs[b], out_hbm.at[pl.ds((r0 + j) * GCH, GCH)])
            return 0

        lax.fori_loop(0, G_PER_W, body, 0)

    return k(table, idx3d)


def _sc_scatter_add(m2, idx3d, zeros):
    mesh = plsc.VectorSubcoreMesh(core_axis_name="c", subcore_axis_name="s")

    @functools.partial(
        pl.kernel,
        out_type=jax.ShapeDtypeStruct((NC, N_PAD, H), jnp.float32),
        mesh=mesh,
        scratch_types=[
            pltpu.VMEM((S_PER_W, SCH), jnp.int32),
            pltpu.VMEM((SCH, H), jnp.float32),
            pltpu.VMEM((SCH, H), jnp.float32),
            pltpu.VMEM_SHARED((N_PAD, H), jnp.float32),
            pltpu.SemaphoreType.DMA,
            pltpu.SemaphoreType.DMA,
        ],
        compiler_params=pltpu.CompilerParams(use_tc_tiling_on_sc=False),
    )
    def k(m2_hbm, idx_hbm, zeros_hbm, out_hbm, idx_v, buf0, buf1, acc_sh,
          sem0, sem1):
        c = lax.axis_index("c")
        s = lax.axis_index("s")
        w = s * NC + c
        pltpu.sync_copy(zeros_hbm.at[pl.ds(s * N_PER_S, N_PER_S)],
                        acc_sh.at[pl.ds(s * N_PER_S, N_PER_S)])
        plsc.subcore_barrier()
        r0 = w * S_PER_W
        pltpu.sync_copy(idx_hbm.at[w], idx_v)
        bufs = (buf0, buf1)
        sems = (sem0, sem1)
        pltpu.async_copy(m2_hbm.at[pl.ds(r0 * SCH, SCH)], buf0, sem0)

        def body(j, _):
            @pl.when(j + 1 < S_PER_W)
            def _():
                for b in range(2):
                    @pl.when((j + 1) % 2 == b)
                    def _():
                        pltpu.async_copy(
                            m2_hbm.at[pl.ds((r0 + j + 1) * SCH, SCH)],
                            bufs[b], sems[b])
            for b in range(2):
                @pl.when(j % 2 == b)
                def _():
                    pltpu.make_async_copy(
                        m2_hbm.at[pl.ds((r0 + j) * SCH, SCH)],
                        bufs[b], sems[b]).wait()
                    pltpu.sync_copy(bufs[b], acc_sh.at[idx_v.at[j]],
                                    add=True)
            return 0

        lax.fori_loop(0, S_PER_W, body, 0)
        plsc.subcore_barrier()
        pltpu.sync_copy(acc_sh.at[pl.ds(s * N_PER_S, N_PER_S)],
                        out_hbm.at[c, pl.ds(s * N_PER_S, N_PER_S)])

    return k(m2, idx3d, zeros)


def _enc_body(a_ref, w_ref, v_ref, o_ref):
    a = a_ref[...]
    u = jnp.dot(a, w_ref[...], preferred_element_type=jnp.float32) + v_ref[0]
    o_ref[...] = _act(_ln(u, v_ref[1], v_ref[2]))


def _tc_encode(a, w, vecs, blk):
    n = a.shape[0]
    return pl.pallas_call(
        _enc_body,
        grid=(n // blk,),
        in_specs=[
            pl.BlockSpec((blk, a.shape[1]), lambda i: (i, 0)),
            pl.BlockSpec(w.shape, lambda i: (0, 0)),
            pl.BlockSpec(vecs.shape, lambda i: (0, 0)),
        ],
        out_specs=pl.BlockSpec((blk, H), lambda i: (i, 0)),
        out_shape=jax.ShapeDtypeStruct((n, H), jnp.float32),
    )(a, w, vecs)


def _edge_body(with_msg, xs_ref, xd_ref, e_ref, wa_ref, wb_ref, v_ref,
               s_ref, *m2_ref):
    v = v_ref[...]
    hin = jnp.concatenate([xs_ref[...], xd_ref[...], e_ref[...]], axis=1)
    u = jnp.dot(hin, wa_ref[...], preferred_element_type=jnp.float32)
    h1 = _act(_ln(u[:, :H] + v[0], v[2], v[3]))
    if with_msg:
        m = _act(u[:, H:] + v[1])
        z = jnp.dot(jnp.concatenate([h1, m], axis=1), wb_ref[...],
                    preferred_element_type=jnp.float32)
        z1 = z[:, :H]
    else:
        z1 = jnp.dot(h1, wb_ref[...], preferred_element_type=jnp.float32)
    h2 = _act(_ln(z1 + v[6], v[4], v[5]))
    s = jnp.sum(h2 * v[8], axis=1) + v_ref[9, 0]
    s_ref[...] = s[None, None, :]
    if with_msg:
        m2_ref[0][...] = (z[:, H:] + v[7]) * jax.nn.sigmoid(s)[:, None]


def _tc_edge(gcat, e, half, wa, wb, vecs, with_msg):
    kin = 3 * H
    wdim = 2 * H if with_msg else H
    off = half * NB
    out_shape = [jax.ShapeDtypeStruct((NB, 1, BE), jnp.float32)]
    out_specs = [pl.BlockSpec((1, 1, BE), lambda i: (i, 0, 0))]
    if with_msg:
        out_shape.append(jax.ShapeDtypeStruct((E2, H), jnp.float32))
        out_specs.append(pl.BlockSpec((BE, H), lambda i: (i, 0)))
    outs = pl.pallas_call(
        functools.partial(_edge_body, with_msg),
        grid=(NB,),
        in_specs=[
            pl.BlockSpec((BE, H), lambda i: (i, 0)),
            pl.BlockSpec((BE, H), lambda i: (i + NB, 0)),
            pl.BlockSpec((BE, H), lambda i: (i + off, 0)),
            pl.BlockSpec((kin, wdim), lambda i: (0, 0)),
            pl.BlockSpec((wdim, wdim), lambda i: (0, 0)),
            pl.BlockSpec((10, H), lambda i: (0, 0)),
        ],
        out_specs=out_specs,
        out_shape=out_shape,
    )(gcat, gcat, e, wa, wb, vecs)
    s = outs[0].reshape(E2)
    return (s, outs[1]) if with_msg else (s, None)


def _node_upd_body(x_ref, a0_ref, a1_ref, a2_ref, a3_ref, w1_ref, w2_ref,
                   v_ref, o_ref):
    x = x_ref[...]
    agg = (a0_ref[0] + a1_ref[0]) + (a2_ref[0] + a3_ref[0])
    u = jnp.dot(jnp.concatenate([x, agg], axis=1), w1_ref[...],
                preferred_element_type=jnp.float32)
    h = _act(_ln(u + v_ref[0], v_ref[1], v_ref[2]))
    o_ref[...] = x + jnp.dot(h, w2_ref[...],
                             preferred_element_type=jnp.float32) + v_ref[3]


def _tc_node_update(x, aggA, aggB, w1, w2, vecs):
    blk = 2000
    return pl.pallas_call(
        _node_upd_body,
        grid=(N // blk,),
        in_specs=[
            pl.BlockSpec((blk, H), lambda i: (i, 0)),
            pl.BlockSpec((1, blk, H), lambda i: (0, i, 0)),
            pl.BlockSpec((1, blk, H), lambda i: (1, i, 0)),
            pl.BlockSpec((1, blk, H), lambda i: (0, i, 0)),
            pl.BlockSpec((1, blk, H), lambda i: (1, i, 0)),
            pl.BlockSpec((2 * H, H), lambda i: (0, 0)),
            pl.BlockSpec((H, H), lambda i: (0, 0)),
            pl.BlockSpec(vecs.shape, lambda i: (0, 0)),
        ],
        out_specs=pl.BlockSpec((blk, H), lambda i: (i, 0)),
        out_shape=jax.ShapeDtypeStruct((N, H), jnp.float32),
    )(x, aggA, aggA, aggB, aggB, w1, w2, vecs)


def _node_cls_body(x_ref, w1_ref, v_ref, o_ref):
    t = _act(jnp.dot(x_ref[...], w1_ref[...],
                     preferred_element_type=jnp.float32) + v_ref[0, :H // 2])
    o_ref[...] = (jnp.sum(t * v_ref[1, :H // 2], axis=1)
                  + v_ref[2, 0])[None, None, :]


def _tc_node_cls(x, w1, vecs):
    blk = 2000
    return pl.pallas_call(
        _node_cls_body,
        grid=(N // blk,),
        in_specs=[
            pl.BlockSpec((blk, H), lambda i: (i, 0)),
            pl.BlockSpec((H, H // 2), lambda i: (0, 0)),
            pl.BlockSpec(vecs.shape, lambda i: (0, 0)),
        ],
        out_specs=pl.BlockSpec((1, 1, blk), lambda i: (i, 0, 0)),
        out_shape=jax.ShapeDtypeStruct((N // blk, 1, blk), jnp.float32),
    )(x, w1, vecs).reshape(N)


def _edge_weights(ep, npp):
    if npp is not None:
        wa = jnp.zeros((3 * H, 2 * H), jnp.float32)
        wa = wa.at[:, :H].set(ep['W1'])
        wa = wa.at[0:H, H:].set(npp['mW1'][0:H])
        wa = wa.at[2 * H:3 * H, H:].set(npp['mW1'][H:2 * H])
        wb = jnp.zeros((2 * H, 2 * H), jnp.float32)
        wb = wb.at[:H, :H].set(ep['W2']).at[H:, H:].set(npp['mW2'])
        mb1, mb2 = npp['mb1'], npp['mb2']
    else:
        wa = ep['W1']
        wb = ep['W2']
        mb1 = mb2 = jnp.zeros((H,), jnp.float32)
    vecs = jnp.stack([
        ep['b1'], mb1, ep['g1'], ep['bb1'], ep['g2'], ep['bb2'],
        ep['b2'], mb2, ep['W3'][:, 0],
        jnp.full((H,), ep['b3'][0], jnp.float32),
    ])
    return wa, wb, vecs


def kernel(node_features, edge_index, edge_attr, params):
    p = params
    src = edge_index[0].astype(jnp.int32)
    dst = edge_index[1].astype(jnp.int32)
    idx_g = [
        jnp.concatenate([src[h * E2:(h + 1) * E2], dst[h * E2:(h + 1) * E2]]
                        ).reshape(NW, G_PER_W, GCH)
        for h in range(2)
    ]
    idx_s = [dst[h * E2:(h + 1) * E2].reshape(NW, S_PER_W, SCH)
             for h in range(2)]
    zeros = jnp.zeros((N_PAD, H), jnp.float32)

    ne = p['node_enc']
    x = _tc_encode(node_features, ne['W'],
                   jnp.stack([ne['b'], ne['g'], ne['bb']]), N)
    ee = p['edge_enc']
    e = _tc_encode(edge_attr, ee['W'],
                   jnp.stack([ee['b'], ee['g'], ee['bb']]), BE)

    inter = []
    for i in range(4):
        ep = p['edge_nets'][i]
        npp = p['node_nets'][i]
        wa, wb, vecs = _edge_weights(ep, npp)
        gA = _sc_gather(x, idx_g[0])
        sA, m2A = _tc_edge(gA, e, 0, wa, wb, vecs, True)
        gB = _sc_gather(x, idx_g[1])
        sB, m2B = _tc_edge(gB, e, 1, wa, wb, vecs, True)
        aggA = _sc_scatter_add(m2A, idx_s[0], zeros)
        aggB = _sc_scatter_add(m2B, idx_s[1], zeros)
        inter.append(jnp.concatenate([sA, sB]))
        nvecs = jnp.stack([npp['ub1'], npp['ug1'], npp['ugb1'], npp['ub2']])
        x = _tc_node_update(x, aggA, aggB, npp['uW1'], npp['uW2'], nvecs)

    wa, wb, vecs = _edge_weights(p['edge_cls'], None)
    gA = _sc_gather(x, idx_g[0])
    fesA, _ = _tc_edge(gA, e, 0, wa, wb, vecs, False)
    gB = _sc_gather(x, idx_g[1])
    fesB, _ = _tc_edge(gB, e, 1, wa, wb, vecs, False)
    fes = jnp.concatenate([fesA, fesB])

    nc = p['node_cls']
    cvecs = jnp.zeros((3, H), jnp.float32)
    cvecs = cvecs.at[0, :H // 2].set(nc['b1'])
    cvecs = cvecs.at[1, :H // 2].set(nc['W2'][:, 0])
    cvecs = cvecs.at[2, 0].set(nc['b2'][0])
    ns = _tc_node_cls(x, nc['W1'], cvecs)
    return (fes, ns, inter)
